# Initial kernel scaffold; baseline (speedup 1.0000x reference)
import jax
import jax.numpy as jnp
from jax.experimental import pallas as pl


def kernel(x, Wg, gate_w, up_w, down_w):
    raise NotImplementedError("write your pallas kernel here")


# speedup vs baseline: 1.0656x; 1.0656x over previous
import functools

import jax
import jax.numpy as jnp
from jax import lax
from jax.experimental import pallas as pl
from jax.experimental.pallas import tpu as pltpu

HIDDEN = 1024
INTER = 2048
NUM_EXPERTS = 8
TOP_K = 2
LANES = 128

_BT = 512
_IB = 512


def _router_body(x_ref, wg_ref, cw_ref):
    x = x_ref[...]
    wg = wg_ref[...]
    logits = lax.dot_general(x, wg, (((1,), (1,)), ((), ())),
                             preferred_element_type=jnp.float32)
    lane = lax.broadcasted_iota(jnp.int32, logits.shape, 1)
    neg = jnp.float32(-1e30)
    logits = jnp.where(lane < NUM_EXPERTS, logits, neg)
    m1 = jnp.max(logits, axis=1, keepdims=True)
    i1 = jnp.min(jnp.where(logits == m1, lane, LANES), axis=1, keepdims=True)
    logits2 = jnp.where(lane == i1, neg, logits)
    m2 = jnp.max(logits2, axis=1, keepdims=True)
    i2 = jnp.min(jnp.where(logits2 == m2, lane, LANES), axis=1, keepdims=True)
    t = jnp.exp(m2 - m1)
    w1 = 1.0 / (1.0 + t)
    w2 = 1.0 - w1
    cw_ref[...] = jnp.where(lane == i1, w1, 0.0) + jnp.where(lane == i2, w2, 0.0)


def _dense_body(x_ref, cw_ref, gw_ref, uw_ref, dw_ref, out_ref):
    e = pl.program_id(1)
    n = pl.program_id(2)
    x = x_ref[...]
    g = lax.dot_general(x, gw_ref[0], (((1,), (1,)), ((), ())),
                        preferred_element_type=jnp.float32)
    u = lax.dot_general(x, uw_ref[0], (((1,), (1,)), ((), ())),
                        preferred_element_type=jnp.float32)
    h = g * jax.nn.sigmoid(g) * u
    y = lax.dot_general(h, dw_ref[0], (((1,), (1,)), ((), ())),
                        preferred_element_type=jnp.float32)
    lane = lax.broadcasted_iota(jnp.int32, cw_ref.shape, 1)
    c = jnp.sum(jnp.where(lane == e, cw_ref[...], 0.0), axis=1, keepdims=True)
    val = c * y

    @pl.when((e == 0) & (n == 0))
    def _():
        out_ref[...] = val

    @pl.when((e > 0) | (n > 0))
    def _():
        out_ref[...] += val


def kernel(x, Wg, gate_w, up_w, down_w):
    batch, seq, hidden = x.shape
    tokens = batch * seq
    xf = x.reshape(tokens, hidden)
    wg_pad = jnp.zeros((LANES, hidden), Wg.dtype).at[:NUM_EXPERTS].set(Wg)

    combine = pl.pallas_call(
        _router_body,
        out_shape=jax.ShapeDtypeStruct((tokens, LANES), jnp.float32),
    )(xf, wg_pad)

    nt = tokens // _BT
    ni = INTER // _IB
    out = pl.pallas_call(
        _dense_body,
        grid=(nt, NUM_EXPERTS, ni),
        in_specs=[
            pl.BlockSpec((_BT, hidden), lambda i, e, n: (i, 0)),
            pl.BlockSpec((_BT, LANES), lambda i, e, n: (i, 0)),
            pl.BlockSpec((1, _IB, hidden), lambda i, e, n: (e, n, 0)),
            pl.BlockSpec((1, _IB, hidden), lambda i, e, n: (e, n, 0)),
            pl.BlockSpec((1, hidden, _IB), lambda i, e, n: (e, 0, n)),
        ],
        out_specs=pl.BlockSpec((_BT, hidden), lambda i, e, n: (i, 0)),
        out_shape=jax.ShapeDtypeStruct((tokens, hidden), jnp.float32),
    )(xf, combine, gate_w, up_w, down_w)
    return out.reshape(batch, seq, hidden)
